# baseline (speedup 1.0000x reference)
import functools

import jax
import jax.numpy as jnp
from jax import lax
from jax.experimental import pallas as pl
from jax.experimental.pallas import tpu as pltpu
from jax.experimental.pallas import tpu_sc as plsc

B = 16384
D = 32
G = 128 // D
NC = 2
NS = 16
NW = NC * NS
BPW = B // NW
CHUNK = 128
NCHUNK = BPW // CHUNK
L = 16


@functools.lru_cache(maxsize=None)
def _make_sc_gather():
    mesh = plsc.VectorSubcoreMesh(
        core_axis_name="c", subcore_axis_name="s", num_cores=NC, num_subcores=NS
    )

    @functools.partial(
        pl.kernel,
        out_type=jax.ShapeDtypeStruct((4, B, 128), jnp.float32),
        mesh=mesh,
        scratch_types=[
            pltpu.VMEM((NCHUNK, CHUNK), jnp.int32),
            pltpu.VMEM((NCHUNK, CHUNK), jnp.int32),
            pltpu.VMEM((CHUNK, 128), jnp.float32),
            pltpu.VMEM((CHUNK, 128), jnp.float32),
            pltpu.VMEM((CHUNK, 128), jnp.float32),
            pltpu.VMEM((CHUNK, 128), jnp.float32),
            pltpu.SemaphoreType.DMA,
            pltpu.SemaphoreType.DMA,
        ],
    )
    def _sc_gather(uid_h, jid_h, umt_h, jmt_h, ugt_h, jgt_h, out,
                   uq, jq, bum, bjm, bug, bjg, sem, wsem):
        wid = lax.axis_index("s") * NC + lax.axis_index("c")
        r0 = wid * NCHUNK
        pltpu.sync_copy(uid_h.at[pl.ds(r0, NCHUNK)], uq)
        pltpu.sync_copy(jid_h.at[pl.ds(r0, NCHUNK)], jq)

        for c in range(NCHUNK):
            def shift(g, _, c=c):
                s = pl.ds(g * L, L)
                uq[c, s] = lax.shift_right_logical(uq[c, s], 2)
                jq[c, s] = lax.shift_right_logical(jq[c, s], 2)
                return ()
            lax.fori_loop(0, CHUNK // L, shift, ())

        base = wid * BPW
        for c in range(NCHUNK):
            if c > 0:
                prev = pl.ds(base + (c - 1) * CHUNK, CHUNK)
                pltpu.make_async_copy(bum, out.at[0, prev], wsem).wait()
                pltpu.make_async_copy(bjm, out.at[1, prev], wsem).wait()
                pltpu.make_async_copy(bug, out.at[2, prev], wsem).wait()
                pltpu.make_async_copy(bjg, out.at[3, prev], wsem).wait()

            cps = [
                pltpu.async_copy(umt_h.at[uq.at[c]], bum, sem),
                pltpu.async_copy(jmt_h.at[jq.at[c]], bjm, sem),
                pltpu.async_copy(ugt_h.at[uq.at[c]], bug, sem),
                pltpu.async_copy(jgt_h.at[jq.at[c]], bjg, sem),
            ]
            for cp in cps:
                cp.wait()

            dst = pl.ds(base + c * CHUNK, CHUNK)
            pltpu.async_copy(bum, out.at[0, dst], wsem)
            pltpu.async_copy(bjm, out.at[1, dst], wsem)
            pltpu.async_copy(bug, out.at[2, dst], wsem)
            pltpu.async_copy(bjg, out.at[3, dst], wsem)

        last = pl.ds(base + (NCHUNK - 1) * CHUNK, CHUNK)
        pltpu.make_async_copy(bum, out.at[0, last], wsem).wait()
        pltpu.make_async_copy(bjm, out.at[1, last], wsem).wait()
        pltpu.make_async_copy(bug, out.at[2, last], wsem).wait()
        pltpu.make_async_copy(bjg, out.at[3, last], wsem).wait()

    return _sc_gather


BLK = 1024


def _select(x, sel):
    acc = sel[0] * x[:, 0 * D:1 * D]
    for m in range(1, G):
        acc = acc + sel[m] * x[:, m * D:(m + 1) * D]
    return acc


def _tc_body(g, uid, jid, w1, b1, w2, b2, w3, b3, w4, w5, b4, b5, out):
    mu = uid[:] & (G - 1)
    mj = jid[:] & (G - 1)
    usel = [(mu == m).astype(jnp.float32) for m in range(G)]
    jsel = [(mj == m).astype(jnp.float32) for m in range(G)]
    um = _select(g[0], usel)
    jm = _select(g[1], jsel)
    ug = _select(g[2], usel)
    jg = _select(g[3], jsel)
    w1v = w1[:]
    x = jnp.maximum(um @ w1v[:D, :] + jm @ w1v[D:, :] + b1[:], 0.0)
    x = jnp.maximum(x @ w2[:] + b2[:], 0.0)
    x = jnp.maximum(x @ w3[:] + b3[:], 0.0)
    x = jnp.maximum(x @ w4[:] + b4[0], 0.0)
    dot = jnp.sum(ug * jg, axis=1, keepdims=True)
    su = jnp.sum(ug * ug, axis=1, keepdims=True)
    sj = jnp.sum(jg * jg, axis=1, keepdims=True)
    gmf = dot * lax.rsqrt(jnp.maximum(su, 1e-12)) * lax.rsqrt(jnp.maximum(sj, 1e-12))
    out[:] = x * w5[0, 0] + gmf * w5[1, 0] + b5[0]


def _tc_dense(g, uid, jid, w1, b1, w2, b2, w3, b3, w4, w5, b4, b5):
    full = lambda a: pl.BlockSpec(a.shape, lambda i, _n=a.ndim: (0,) * _n)
    smem = pl.BlockSpec(memory_space=pltpu.SMEM)
    idspec = pl.BlockSpec((BLK, 1), lambda i: (i, 0))
    return pl.pallas_call(
        _tc_body,
        grid=(B // BLK,),
        in_specs=[pl.BlockSpec((4, BLK, 128), lambda i: (0, i, 0)),
                  idspec, idspec,
                  full(w1), full(b1), full(w2), full(b2), full(w3), full(b3),
                  full(w4), smem, smem, smem],
        out_specs=pl.BlockSpec((BLK, 1), lambda i: (i, 0)),
        out_shape=jax.ShapeDtypeStruct((B, 1), jnp.float32),
    )(g, uid, jid, w1, b1, w2, b2, w3, b3, w4, w5, b4, b5)


def kernel(user_ids, joke_ids, user_mlp_table, joke_mlp_table,
           user_gmf_table, joke_gmf_table,
           W1, b1, W2, b2, W3, b3, W4, b4, W5, b5):
    uid = user_ids.astype(jnp.int32)
    jid = joke_ids.astype(jnp.int32)
    umt = user_mlp_table[:-1].reshape(-1, 128)
    ugt = user_gmf_table[:-1].reshape(-1, 128)
    jmt = joke_mlp_table[:-1].reshape(-1, 128)
    jgt = joke_gmf_table[:-1].reshape(-1, 128)
    g = _make_sc_gather()(uid.reshape(B // CHUNK, CHUNK),
                          jid.reshape(B // CHUNK, CHUNK),
                          umt, jmt, ugt, jgt)
    return _tc_dense(g, uid.reshape(B, 1), jid.reshape(B, 1),
                     W1, b1, W2, b2, W3, b3, W4, W5, b4, b5)


# speedup vs baseline: 1.0280x; 1.0280x over previous
import functools

import jax
import jax.numpy as jnp
from jax import lax
from jax.experimental import pallas as pl
from jax.experimental.pallas import tpu as pltpu
from jax.experimental.pallas import tpu_sc as plsc

B = 16384
D = 32
NC = 2
NS = 16
NW = NC * NS
BPW = B // NW
CHUNK = 128
NCHUNK = BPW // CHUNK
L = 16


@functools.lru_cache(maxsize=None)
def _make_sc_gather():
    mesh = plsc.VectorSubcoreMesh(
        core_axis_name="c", subcore_axis_name="s", num_cores=NC, num_subcores=NS
    )

    @functools.partial(
        pl.kernel,
        out_type=jax.ShapeDtypeStruct((4 * B * D,), jnp.float32),
        mesh=mesh,
        scratch_types=[
            pltpu.VMEM((BPW,), jnp.int32),
            pltpu.VMEM((BPW,), jnp.int32),
            pltpu.VMEM((CHUNK * D,), jnp.float32),
            pltpu.VMEM((CHUNK * D,), jnp.float32),
            pltpu.VMEM((CHUNK * D,), jnp.float32),
            pltpu.VMEM((CHUNK * D,), jnp.float32),
            pltpu.SemaphoreType.DMA,
            pltpu.SemaphoreType.DMA,
        ],
        compiler_params=pltpu.CompilerParams(use_tc_tiling_on_sc=False),
    )
    def _sc_gather(uid_h, jid_h, umt_h, jmt_h, ugt_h, jgt_h, out,
                   uidx, jidx, bum, bjm, bug, bjg, sem, wsem):
        wid = lax.axis_index("s") * NC + lax.axis_index("c")
        base = wid * BPW
        pltpu.sync_copy(uid_h.at[pl.ds(base, BPW)], uidx)
        pltpu.sync_copy(jid_h.at[pl.ds(base, BPW)], jidx)

        bufs = (bum, bjm, bug, bjg)
        for c in range(NCHUNK):
            if c > 0:
                for t in range(4):
                    pltpu.make_async_copy(
                        bufs[t], out.at[pl.ds(0, CHUNK * D)], wsem).wait()

            def issue(g, _, c=c):
                vu = uidx[pl.ds(c * CHUNK + g * L, L)]
                vj = jidx[pl.ds(c * CHUNK + g * L, L)]
                for i in range(L):
                    u = vu[i]
                    j = vj[i]
                    k = (g * L + i) * D
                    pltpu.async_copy(umt_h.at[pl.ds(u * D, D)], bum.at[pl.ds(k, D)], sem)
                    pltpu.async_copy(jmt_h.at[pl.ds(j * D, D)], bjm.at[pl.ds(k, D)], sem)
                    pltpu.async_copy(ugt_h.at[pl.ds(u * D, D)], bug.at[pl.ds(k, D)], sem)
                    pltpu.async_copy(jgt_h.at[pl.ds(j * D, D)], bjg.at[pl.ds(k, D)], sem)
                return ()
            lax.fori_loop(0, CHUNK // L, issue, ())

            def drain(k, _):
                pltpu.make_async_copy(umt_h.at[pl.ds(0, D)], bum.at[pl.ds(0, D)], sem).wait()
                pltpu.make_async_copy(jmt_h.at[pl.ds(0, D)], bjm.at[pl.ds(0, D)], sem).wait()
                pltpu.make_async_copy(ugt_h.at[pl.ds(0, D)], bug.at[pl.ds(0, D)], sem).wait()
                pltpu.make_async_copy(jgt_h.at[pl.ds(0, D)], bjg.at[pl.ds(0, D)], sem).wait()
                return ()
            lax.fori_loop(0, CHUNK, drain, ())

            for t in range(4):
                dst = pl.ds((t * B + base + c * CHUNK) * D, CHUNK * D)
                pltpu.async_copy(bufs[t], out.at[dst], wsem)

        for t in range(4):
            pltpu.make_async_copy(bufs[t], out.at[pl.ds(0, CHUNK * D)], wsem).wait()

    return _sc_gather


BLK = 2048


def _tc_body(g, w1, b1, w2, b2, w3, b3, w4, w5, b4, b5, out):
    um = g[0]
    jm = g[1]
    ug = g[2]
    jg = g[3]
    w1v = w1[:]
    x = jnp.maximum(um @ w1v[:D, :] + jm @ w1v[D:, :] + b1[:], 0.0)
    x = jnp.maximum(x @ w2[:] + b2[:], 0.0)
    x = jnp.maximum(x @ w3[:] + b3[:], 0.0)
    x = jnp.maximum(x @ w4[:] + b4[0], 0.0)
    dot = jnp.sum(ug * jg, axis=1, keepdims=True)
    su = jnp.sum(ug * ug, axis=1, keepdims=True)
    sj = jnp.sum(jg * jg, axis=1, keepdims=True)
    gmf = dot * lax.rsqrt(jnp.maximum(su, 1e-12)) * lax.rsqrt(jnp.maximum(sj, 1e-12))
    out[:] = x * w5[0, 0] + gmf * w5[1, 0] + b5[0]


def _tc_dense(g, w1, b1, w2, b2, w3, b3, w4, w5, b4, b5):
    full = lambda a: pl.BlockSpec(a.shape, lambda i, _n=a.ndim: (0,) * _n)
    smem = pl.BlockSpec(memory_space=pltpu.SMEM)
    return pl.pallas_call(
        _tc_body,
        grid=(B // BLK,),
        in_specs=[pl.BlockSpec((4, BLK, D), lambda i: (0, i, 0)),
                  full(w1), full(b1), full(w2), full(b2), full(w3), full(b3),
                  full(w4), smem, smem, smem],
        out_specs=pl.BlockSpec((BLK, 1), lambda i: (i, 0)),
        out_shape=jax.ShapeDtypeStruct((B, 1), jnp.float32),
    )(g, w1, b1, w2, b2, w3, b3, w4, w5, b4, b5)


def kernel(user_ids, joke_ids, user_mlp_table, joke_mlp_table,
           user_gmf_table, joke_gmf_table,
           W1, b1, W2, b2, W3, b3, W4, b4, W5, b5):
    uid = user_ids.astype(jnp.int32)
    jid = joke_ids.astype(jnp.int32)
    flat = _make_sc_gather()(uid, jid,
                             user_mlp_table.reshape(-1),
                             joke_mlp_table.reshape(-1),
                             user_gmf_table.reshape(-1),
                             joke_gmf_table.reshape(-1))
    g = flat.reshape(4, B, D)
    return _tc_dense(g, W1, b1, W2, b2, W3, b3, W4, W5, b4, b5)


# speedup vs baseline: 1.4755x; 1.4353x over previous
import functools

import jax
import jax.numpy as jnp
from jax import lax
from jax.experimental import pallas as pl
from jax.experimental.pallas import tpu as pltpu
from jax.experimental.pallas import tpu_sc as plsc

B = 16384
D = 32
NC = 2
NS = 16
NW = NC * NS
BPW = B // NW
CHUNK = 128
NCHUNK = BPW // CHUNK
L = 16


@functools.lru_cache(maxsize=None)
def _make_sc_gather():
    mesh = plsc.VectorSubcoreMesh(
        core_axis_name="c", subcore_axis_name="s", num_cores=NC, num_subcores=NS
    )

    @functools.partial(
        pl.kernel,
        out_type=jax.ShapeDtypeStruct((4, B, D), jnp.float32),
        mesh=mesh,
        scratch_types=[
            pltpu.VMEM((BPW,), jnp.int32),
            pltpu.VMEM((BPW,), jnp.int32),
            pltpu.VMEM((CHUNK, D), jnp.float32),
            pltpu.VMEM((CHUNK, D), jnp.float32),
            pltpu.VMEM((CHUNK, D), jnp.float32),
            pltpu.VMEM((CHUNK, D), jnp.float32),
            pltpu.SemaphoreType.DMA,
            pltpu.SemaphoreType.DMA,
        ],
        compiler_params=pltpu.CompilerParams(needs_layout_passes=True),
    )
    def _sc_gather(uid_h, jid_h, umt_h, jmt_h, ugt_h, jgt_h, out,
                   uidx, jidx, bum, bjm, bug, bjg, sem, wsem):
        wid = lax.axis_index("s") * NC + lax.axis_index("c")
        base = wid * BPW
        pltpu.sync_copy(uid_h.at[pl.ds(base, BPW)], uidx)
        pltpu.sync_copy(jid_h.at[pl.ds(base, BPW)], jidx)

        for c in range(NCHUNK):
            if c > 0:
                pltpu.make_async_copy(bum, out.at[0, pl.ds(base, CHUNK)], wsem).wait()
                pltpu.make_async_copy(bjm, out.at[1, pl.ds(base, CHUNK)], wsem).wait()
                pltpu.make_async_copy(bug, out.at[2, pl.ds(base, CHUNK)], wsem).wait()
                pltpu.make_async_copy(bjg, out.at[3, pl.ds(base, CHUNK)], wsem).wait()

            def issue(g, _, c=c):
                vu = uidx[pl.ds(c * CHUNK + g * L, L)]
                vj = jidx[pl.ds(c * CHUNK + g * L, L)]
                for i in range(L):
                    u = vu[i]
                    j = vj[i]
                    k = g * L + i
                    pltpu.async_copy(umt_h.at[pl.ds(u, 1)], bum.at[pl.ds(k, 1)], sem)
                    pltpu.async_copy(jmt_h.at[pl.ds(j, 1)], bjm.at[pl.ds(k, 1)], sem)
                    pltpu.async_copy(ugt_h.at[pl.ds(u, 1)], bug.at[pl.ds(k, 1)], sem)
                    pltpu.async_copy(jgt_h.at[pl.ds(j, 1)], bjg.at[pl.ds(k, 1)], sem)
                return ()
            lax.fori_loop(0, CHUNK // L, issue, ())

            def drain(k, _):
                pltpu.make_async_copy(umt_h.at[pl.ds(0, 1)], bum.at[pl.ds(0, 1)], sem).wait()
                pltpu.make_async_copy(jmt_h.at[pl.ds(0, 1)], bjm.at[pl.ds(0, 1)], sem).wait()
                pltpu.make_async_copy(ugt_h.at[pl.ds(0, 1)], bug.at[pl.ds(0, 1)], sem).wait()
                pltpu.make_async_copy(jgt_h.at[pl.ds(0, 1)], bjg.at[pl.ds(0, 1)], sem).wait()
                return ()
            lax.fori_loop(0, CHUNK, drain, ())

            dst = pl.ds(base + c * CHUNK, CHUNK)
            pltpu.async_copy(bum, out.at[0, dst], wsem)
            pltpu.async_copy(bjm, out.at[1, dst], wsem)
            pltpu.async_copy(bug, out.at[2, dst], wsem)
            pltpu.async_copy(bjg, out.at[3, dst], wsem)

        pltpu.make_async_copy(bum, out.at[0, pl.ds(base, CHUNK)], wsem).wait()
        pltpu.make_async_copy(bjm, out.at[1, pl.ds(base, CHUNK)], wsem).wait()
        pltpu.make_async_copy(bug, out.at[2, pl.ds(base, CHUNK)], wsem).wait()
        pltpu.make_async_copy(bjg, out.at[3, pl.ds(base, CHUNK)], wsem).wait()

    return _sc_gather


BLK = 2048


def _tc_body(g, w1, b1, w2, b2, w3, b3, w4, w5, b4, b5, out):
    um = g[0]
    jm = g[1]
    ug = g[2]
    jg = g[3]
    w1v = w1[:]
    x = jnp.maximum(um @ w1v[:D, :] + jm @ w1v[D:, :] + b1[:], 0.0)
    x = jnp.maximum(x @ w2[:] + b2[:], 0.0)
    x = jnp.maximum(x @ w3[:] + b3[:], 0.0)
    x = jnp.maximum(x @ w4[:] + b4[0], 0.0)
    dot = jnp.sum(ug * jg, axis=1, keepdims=True)
    su = jnp.sum(ug * ug, axis=1, keepdims=True)
    sj = jnp.sum(jg * jg, axis=1, keepdims=True)
    gmf = dot * lax.rsqrt(jnp.maximum(su, 1e-12)) * lax.rsqrt(jnp.maximum(sj, 1e-12))
    out[:] = x * w5[0, 0] + gmf * w5[1, 0] + b5[0]


def _tc_dense(g, w1, b1, w2, b2, w3, b3, w4, w5, b4, b5):
    full = lambda a: pl.BlockSpec(a.shape, lambda i, _n=a.ndim: (0,) * _n)
    smem = pl.BlockSpec(memory_space=pltpu.SMEM)
    return pl.pallas_call(
        _tc_body,
        grid=(B // BLK,),
        in_specs=[pl.BlockSpec((4, BLK, D), lambda i: (0, i, 0)),
                  full(w1), full(b1), full(w2), full(b2), full(w3), full(b3),
                  full(w4), smem, smem, smem],
        out_specs=pl.BlockSpec((BLK, 1), lambda i: (i, 0)),
        out_shape=jax.ShapeDtypeStruct((B, 1), jnp.float32),
    )(g, w1, b1, w2, b2, w3, b3, w4, w5, b4, b5)


def kernel(user_ids, joke_ids, user_mlp_table, joke_mlp_table,
           user_gmf_table, joke_gmf_table,
           W1, b1, W2, b2, W3, b3, W4, b4, W5, b5):
    uid = user_ids.astype(jnp.int32)
    jid = joke_ids.astype(jnp.int32)
    g = _make_sc_gather()(uid, jid, user_mlp_table, joke_mlp_table,
                          user_gmf_table, joke_gmf_table)
    return _tc_dense(g, W1, b1, W2, b2, W3, b3, W4, W5, b4, b5)


# speedup vs baseline: 1.4766x; 1.0008x over previous
import functools

import jax
import jax.numpy as jnp
from jax import lax
from jax.experimental import pallas as pl
from jax.experimental.pallas import tpu as pltpu
from jax.experimental.pallas import tpu_sc as plsc

B = 16384
D = 32
NC = 2
NS = 16
NW = NC * NS
BPW = B // NW
CHUNK = 128
NCHUNK = BPW // CHUNK
L = 16


@functools.lru_cache(maxsize=None)
def _make_sc_gather():
    mesh = plsc.VectorSubcoreMesh(
        core_axis_name="c", subcore_axis_name="s", num_cores=NC, num_subcores=NS
    )

    @functools.partial(
        pl.kernel,
        out_type=jax.ShapeDtypeStruct((4, B, D), jnp.float32),
        mesh=mesh,
        scratch_types=[
            pltpu.VMEM((BPW,), jnp.int32),
            pltpu.VMEM((BPW,), jnp.int32),
            pltpu.VMEM((CHUNK, D), jnp.float32),
            pltpu.VMEM((CHUNK, D), jnp.float32),
            pltpu.VMEM((CHUNK, D), jnp.float32),
            pltpu.VMEM((CHUNK, D), jnp.float32),
            pltpu.SemaphoreType.DMA,
            pltpu.SemaphoreType.DMA,
        ],
    )
    def _sc_gather(uid_h, jid_h, umt_h, jmt_h, ugt_h, jgt_h, out,
                   uidx, jidx, bum, bjm, bug, bjg, sem, wsem):
        wid = lax.axis_index("s") * NC + lax.axis_index("c")
        base = wid * BPW
        pltpu.sync_copy(uid_h.at[pl.ds(base, BPW)], uidx)
        pltpu.sync_copy(jid_h.at[pl.ds(base, BPW)], jidx)

        for c in range(NCHUNK):
            if c > 0:
                pltpu.make_async_copy(bum, out.at[0, pl.ds(base, CHUNK)], wsem).wait()
                pltpu.make_async_copy(bjm, out.at[1, pl.ds(base, CHUNK)], wsem).wait()
                pltpu.make_async_copy(bug, out.at[2, pl.ds(base, CHUNK)], wsem).wait()
                pltpu.make_async_copy(bjg, out.at[3, pl.ds(base, CHUNK)], wsem).wait()

            def issue(g, _, c=c):
                vu = uidx[pl.ds(c * CHUNK + g * L, L)]
                vj = jidx[pl.ds(c * CHUNK + g * L, L)]
                for i in range(L):
                    u = vu[i]
                    j = vj[i]
                    k = g * L + i
                    pltpu.async_copy(umt_h.at[pl.ds(u, 1)], bum.at[pl.ds(k, 1)], sem)
                    pltpu.async_copy(jmt_h.at[pl.ds(j, 1)], bjm.at[pl.ds(k, 1)], sem)
                    pltpu.async_copy(ugt_h.at[pl.ds(u, 1)], bug.at[pl.ds(k, 1)], sem)
                    pltpu.async_copy(jgt_h.at[pl.ds(j, 1)], bjg.at[pl.ds(k, 1)], sem)
                return ()
            lax.fori_loop(0, CHUNK // L, issue, ())

            def drain(k, _):
                pltpu.make_async_copy(umt_h.at[pl.ds(0, 1)], bum.at[pl.ds(0, 1)], sem).wait()
                pltpu.make_async_copy(jmt_h.at[pl.ds(0, 1)], bjm.at[pl.ds(0, 1)], sem).wait()
                pltpu.make_async_copy(ugt_h.at[pl.ds(0, 1)], bug.at[pl.ds(0, 1)], sem).wait()
                pltpu.make_async_copy(jgt_h.at[pl.ds(0, 1)], bjg.at[pl.ds(0, 1)], sem).wait()
                return ()
            lax.fori_loop(0, CHUNK, drain, ())

            dst = pl.ds(base + c * CHUNK, CHUNK)
            pltpu.async_copy(bum, out.at[0, dst], wsem)
            pltpu.async_copy(bjm, out.at[1, dst], wsem)
            pltpu.async_copy(bug, out.at[2, dst], wsem)
            pltpu.async_copy(bjg, out.at[3, dst], wsem)

        pltpu.make_async_copy(bum, out.at[0, pl.ds(base, CHUNK)], wsem).wait()
        pltpu.make_async_copy(bjm, out.at[1, pl.ds(base, CHUNK)], wsem).wait()
        pltpu.make_async_copy(bug, out.at[2, pl.ds(base, CHUNK)], wsem).wait()
        pltpu.make_async_copy(bjg, out.at[3, pl.ds(base, CHUNK)], wsem).wait()

    return _sc_gather


BLK = 2048


def _tc_body(g, w1, b1, w2, b2, w3, b3, w4, w5, b4, b5, out):
    um = g[0]
    jm = g[1]
    ug = g[2]
    jg = g[3]
    w1v = w1[:]
    x = jnp.maximum(um @ w1v[:D, :] + jm @ w1v[D:, :] + b1[:], 0.0)
    x = jnp.maximum(x @ w2[:] + b2[:], 0.0)
    x = jnp.maximum(x @ w3[:] + b3[:], 0.0)
    x = jnp.maximum(x @ w4[:] + b4[0], 0.0)
    dot = jnp.sum(ug * jg, axis=1, keepdims=True)
    su = jnp.sum(ug * ug, axis=1, keepdims=True)
    sj = jnp.sum(jg * jg, axis=1, keepdims=True)
    gmf = dot * lax.rsqrt(jnp.maximum(su, 1e-12)) * lax.rsqrt(jnp.maximum(sj, 1e-12))
    out[:] = x * w5[0, 0] + gmf * w5[1, 0] + b5[0]


def _tc_dense(g, w1, b1, w2, b2, w3, b3, w4, w5, b4, b5):
    full = lambda a: pl.BlockSpec(a.shape, lambda i, _n=a.ndim: (0,) * _n)
    smem = pl.BlockSpec(memory_space=pltpu.SMEM)
    return pl.pallas_call(
        _tc_body,
        grid=(B // BLK,),
        in_specs=[pl.BlockSpec((4, BLK, D), lambda i: (0, i, 0)),
                  full(w1), full(b1), full(w2), full(b2), full(w3), full(b3),
                  full(w4), smem, smem, smem],
        out_specs=pl.BlockSpec((BLK, 1), lambda i: (i, 0)),
        out_shape=jax.ShapeDtypeStruct((B, 1), jnp.float32),
    )(g, w1, b1, w2, b2, w3, b3, w4, w5, b4, b5)


def kernel(user_ids, joke_ids, user_mlp_table, joke_mlp_table,
           user_gmf_table, joke_gmf_table,
           W1, b1, W2, b2, W3, b3, W4, b4, W5, b5):
    uid = user_ids.astype(jnp.int32)
    jid = joke_ids.astype(jnp.int32)
    g = _make_sc_gather()(uid, jid, user_mlp_table, joke_mlp_table,
                          user_gmf_table, joke_gmf_table)
    return _tc_dense(g, W1, b1, W2, b2, W3, b3, W4, W5, b4, b5)
